# baseline (speedup 1.0000x reference)
import functools

import jax
import jax.numpy as jnp
from jax import lax
from jax.experimental import pallas as pl
from jax.experimental.pallas import tpu as pltpu
from jax.experimental.pallas import tpu_sc as plsc

N = 10000
E = 320000
H = 128
R = 32
G = 64

NC = 2
NS = 16
NW = NC * NS

NP = 10240
CH = 80
GP = 128

_MESH = dict(core_axis_name="c", subcore_axis_name="s")


def _wid():
    return lax.axis_index("s") * NC + lax.axis_index("c")


def _make_gather(B, W):
    assert B % (NW * CH) == 0
    chunks = B // (NW * CH)
    per_w = chunks * CH
    assert chunks % 2 == 0
    nh = chunks // 2

    @functools.partial(
        pl.kernel,
        out_type=jax.ShapeDtypeStruct((B, W), jnp.int32),
        mesh=plsc.VectorSubcoreMesh(**_MESH),
        compiler_params=pltpu.CompilerParams(use_tc_tiling_on_sc=False),
        scratch_types=[
            pltpu.VMEM((per_w,), jnp.int32),
            pltpu.VMEM((CH, W), jnp.int32),
            pltpu.VMEM((CH, W), jnp.int32),
            pltpu.SemaphoreType.DMA,
            pltpu.SemaphoreType.DMA,
        ],
    )
    def gather_k(table, idx, out, idx_v, rb0, rb1, s0, s1):
        base = _wid() * per_w
        pltpu.sync_copy(idx.at[pl.ds(base, per_w)], idx_v)

        def start(i, rb, s):
            pltpu.async_copy(table.at[idx_v.at[pl.ds(i * CH, CH)]], rb, s)

        def wait(rb, s):
            pltpu.make_async_copy(
                table.at[idx_v.at[pl.ds(0, CH)]], rb, s
            ).wait()

        start(0, rb0, s0)

        @pl.loop(0, nh)
        def _(j):
            i0 = 2 * j
            start(i0 + 1, rb1, s1)
            wait(rb0, s0)
            pltpu.sync_copy(rb0, out.at[pl.ds(base + i0 * CH, CH)])

            @pl.when(j < nh - 1)
            def _():
                start(i0 + 2, rb0, s0)

            wait(rb1, s1)
            pltpu.sync_copy(rb1, out.at[pl.ds(base + (i0 + 1) * CH, CH)])

    return gather_k


def _make_scatter(B, nseg):
    assert B % (NW * CH) == 0 and nseg % (NS * 8) == 0
    chunks = B // (NW * CH)
    per_w = chunks * CH
    nh = chunks // 2
    rem = chunks % 2
    rpt = nseg // NS

    @functools.partial(
        pl.kernel,
        out_type=jax.ShapeDtypeStruct((NC, nseg, H), jnp.float32),
        mesh=plsc.VectorSubcoreMesh(**_MESH),
        scratch_types=[
            pltpu.VMEM((CH,), jnp.int32),
            pltpu.VMEM((CH,), jnp.int32),
            pltpu.VMEM((CH, H), jnp.float32),
            pltpu.VMEM((CH, H), jnp.float32),
            pltpu.VMEM_SHARED((nseg, H), jnp.float32),
            pltpu.SemaphoreType.DMA,
            pltpu.SemaphoreType.DMA,
        ],
    )
    def scatter_k(vals, idx, zeros_c, out, ib0, ib1, vb0, vb1, acc, s0, s1):
        cid = lax.axis_index("c")
        sid = lax.axis_index("s")
        rbase = sid * rpt
        pltpu.sync_copy(zeros_c.at[pl.ds(0, CH)], vb0)
        nz = (rpt + CH - 1) // CH
        for j in range(nz):
            rows = min(CH, rpt - j * CH)
            pltpu.sync_copy(
                vb0.at[pl.ds(0, rows)], acc.at[pl.ds(rbase + j * CH, rows)]
            )
        plsc.subcore_barrier()

        base = _wid() * per_w

        def ld_idx(i, ib):
            pltpu.sync_copy(idx.at[pl.ds(base + i * CH, CH)], ib)

        def start(i, vb, s):
            pltpu.async_copy(vals.at[pl.ds(base + i * CH, CH)], vb, s)

        def wait(i, vb, s):
            pltpu.make_async_copy(vals.at[pl.ds(base, CH)], vb, s).wait()

        ld_idx(0, ib0)
        start(0, vb0, s0)

        @pl.loop(0, nh)
        def _(j):
            i0 = 2 * j
            ld_idx(i0 + 1, ib1)
            start(i0 + 1, vb1, s1)
            wait(i0, vb0, s0)
            pltpu.sync_copy(vb0, acc.at[ib0], add=True)

            @pl.when(j < nh - 1)
            def _():
                ld_idx(i0 + 2, ib0)
                start(i0 + 2, vb0, s0)

            wait(i0 + 1, vb1, s1)
            pltpu.sync_copy(vb1, acc.at[ib1], add=True)

        if rem:
            i_last = chunks - 1
            ld_idx(i_last, ib0)
            pltpu.sync_copy(vals.at[pl.ds(base + i_last * CH, CH)], vb0)
            pltpu.sync_copy(vb0, acc.at[ib0], add=True)

        plsc.subcore_barrier()
        pltpu.sync_copy(
            acc.at[pl.ds(rbase, rpt)], out.at[cid, pl.ds(rbase, rpt)]
        )

    return scatter_k


def _make_pool():
    chunks = NP // (NW * CH)
    per_w = chunks * CH
    rpt = GP // NS

    @functools.partial(
        pl.kernel,
        out_type=(
            jax.ShapeDtypeStruct((NC, GP, H), jnp.float32),
            jax.ShapeDtypeStruct((NC, GP, H), jnp.float32),
        ),
        mesh=plsc.VectorSubcoreMesh(**_MESH),
        scratch_types=[
            pltpu.VMEM((CH,), jnp.int32),
            pltpu.VMEM((CH, H), jnp.float32),
            pltpu.VMEM((CH, H), jnp.float32),
            pltpu.VMEM_SHARED((GP, H), jnp.float32),
            pltpu.VMEM_SHARED((GP, H), jnp.float32),
            pltpu.SemaphoreType.DMA,
        ],
    )
    def pool_k(vals, idx, zeros_c, ones_c, out_s, out_n, idx_v, vals_v,
               ones_v, acc_s, acc_n, sem):
        cid = lax.axis_index("c")
        sid = lax.axis_index("s")
        rbase = sid * rpt
        pltpu.sync_copy(zeros_c.at[pl.ds(0, CH)], vals_v)
        pltpu.sync_copy(ones_c.at[pl.ds(0, CH)], ones_v)
        pltpu.sync_copy(vals_v.at[pl.ds(0, rpt)], acc_s.at[pl.ds(rbase, rpt)])
        pltpu.sync_copy(vals_v.at[pl.ds(0, rpt)], acc_n.at[pl.ds(rbase, rpt)])
        plsc.subcore_barrier()

        base = _wid() * per_w

        @pl.loop(0, chunks)
        def _(i):
            off = base + i * CH
            pltpu.sync_copy(idx.at[pl.ds(off, CH)], idx_v)
            pltpu.sync_copy(vals.at[pl.ds(off, CH)], vals_v)
            pltpu.sync_copy(vals_v, acc_s.at[idx_v], add=True)
            pltpu.sync_copy(ones_v, acc_n.at[idx_v], add=True)

        plsc.subcore_barrier()
        pltpu.sync_copy(acc_s.at[pl.ds(rbase, rpt)], out_s.at[cid, pl.ds(rbase, rpt)])
        pltpu.sync_copy(acc_n.at[pl.ds(rbase, rpt)], out_n.at[cid, pl.ds(rbase, rpt)])

    return pool_k


CE = 640


def _edge_tc(hdhs, ea, Wc, bc):
    nb = E // CE

    def body(hd_ref, hs_ref, ea_ref, w_ref, b_ref, m_ref):
        w = w_ref[...]
        acc = jnp.dot(hd_ref[...], w[0:H], preferred_element_type=jnp.float32)
        acc += jnp.dot(hs_ref[...], w[H:2 * H], preferred_element_type=jnp.float32)
        acc += jnp.dot(ea_ref[...], w[2 * H:], preferred_element_type=jnp.float32)
        acc += b_ref[...]
        f = acc[:, :H]
        s = acc[:, H:]
        sig = 1.0 / (1.0 + jnp.exp(-f))
        sp = jnp.maximum(s, 0.0) + jnp.log1p(jnp.exp(-jnp.abs(s)))
        m_ref[...] = sig * sp

    return pl.pallas_call(
        body,
        grid=(nb,),
        in_specs=[
            pl.BlockSpec((CE, H), lambda i: (i, 0)),
            pl.BlockSpec((CE, H), lambda i: (i + nb, 0)),
            pl.BlockSpec((CE, R), lambda i: (i, 0)),
            pl.BlockSpec((2 * H + R, 2 * H), lambda i: (0, 0)),
            pl.BlockSpec((1, 2 * H), lambda i: (0, 0)),
        ],
        out_specs=pl.BlockSpec((CE, H), lambda i: (i, 0)),
        out_shape=jax.ShapeDtypeStruct((E, H), jnp.float32),
    )(hdhs, hdhs, ea, Wc, bc)


CB = 1024


def _combine_tc(h, parts, out_dtype):
    nb = NP // CB

    def body(h_ref, p0_ref, p1_ref, o_ref):
        hv = h_ref[...].astype(jnp.float32)
        o_ref[...] = jnp.maximum(hv + p0_ref[0] + p1_ref[0], 0.0).astype(out_dtype)

    return pl.pallas_call(
        body,
        grid=(nb,),
        in_specs=[
            pl.BlockSpec((CB, H), lambda i: (i, 0)),
            pl.BlockSpec((1, CB, H), lambda i: (0, i, 0)),
            pl.BlockSpec((1, CB, H), lambda i: (1, i, 0)),
        ],
        out_specs=pl.BlockSpec((CB, H), lambda i: (i, 0)),
        out_shape=jax.ShapeDtypeStruct((NP, H), out_dtype),
    )(h, parts, parts)


def _final_tc(sums, cnts, Wl, bl):
    def body(s_ref, c_ref, w_ref, b_ref, o_ref):
        ssum = s_ref[0] + s_ref[1]
        cnt = c_ref[0] + c_ref[1]
        pooled = ssum / jnp.maximum(cnt, 1.0)
        o_ref[...] = (
            jnp.dot(pooled, w_ref[...], preferred_element_type=jnp.float32)
            + b_ref[...]
        )

    return pl.pallas_call(
        body,
        in_specs=[
            pl.BlockSpec((NC, GP, H), lambda: (0, 0, 0)),
            pl.BlockSpec((NC, GP, H), lambda: (0, 0, 0)),
            pl.BlockSpec((H, H), lambda: (0, 0)),
            pl.BlockSpec((1, H), lambda: (0, 0)),
        ],
        out_specs=pl.BlockSpec((GP, H), lambda: (0, 0)),
        out_shape=jax.ShapeDtypeStruct((GP, H), jnp.float32),
    )(sums, cnts, Wl, bl.reshape(1, H))


_gather_emb = _make_gather(NP, H // 2)
_gather_edges = _make_gather(2 * E, H // 2)


def _to_words(t_bf16):
    return lax.bitcast_convert_type(
        t_bf16.reshape(t_bf16.shape[0], H // 2, 2), jnp.int32
    )


def _from_words(t_i32):
    return lax.bitcast_convert_type(t_i32, jnp.bfloat16).reshape(-1, H)
_scatter_edges = _make_scatter(E, NP)
_pool = _make_pool()


def kernel(x, edge_index, edge_attr, batch, emb, Wf1, bf1, Ws1, bs1, Wf2, bf2,
           Ws2, bs2, Wf3, bf3, Ws3, bs3, Wl, bl):
    x = x.astype(jnp.int32)
    src = edge_index[0].astype(jnp.int32)
    dst = edge_index[1].astype(jnp.int32)
    batch = batch.astype(jnp.int32)

    xpad = jnp.pad(x, (0, NP - N))
    bpad = jnp.pad(batch, (0, NP - N), constant_values=G)
    eidx = jnp.concatenate([dst, src])
    zeros_c = jnp.zeros((CH, H), jnp.float32)
    ones_c = jnp.ones((CH, H), jnp.float32)
    ea = edge_attr.astype(jnp.bfloat16)

    hw = _gather_emb(_to_words(emb.astype(jnp.bfloat16)), xpad)
    h = _from_words(hw)

    layers = ((Wf1, bf1, Ws1, bs1), (Wf2, bf2, Ws2, bs2), (Wf3, bf3, Ws3, bs3))
    for li, (Wf, bf, Ws, bs) in enumerate(layers):
        Wc = jnp.concatenate([Wf, Ws], axis=1).astype(jnp.bfloat16)
        bc = jnp.concatenate([bf, bs]).reshape(1, 2 * H)
        hdhs = _from_words(_gather_edges(hw, eidx))
        m = _edge_tc(hdhs, ea, Wc, bc)
        parts = _scatter_edges(m, dst, zeros_c)
        if li == 2:
            h = _combine_tc(h, parts, jnp.float32)
        else:
            h = _combine_tc(h, parts, jnp.bfloat16)
            hw = _to_words(h)

    sums, cnts = _pool(h, bpad, zeros_c, ones_c)
    out = _final_tc(sums, cnts, Wl, bl)
    return out[:G]


# speedup vs baseline: 2.6131x; 2.6131x over previous
import functools

import jax
import jax.numpy as jnp
from jax import lax
from jax.experimental import pallas as pl
from jax.experimental.pallas import tpu as pltpu
from jax.experimental.pallas import tpu_sc as plsc

N = 10000
E = 320000
H = 128
R = 32
G = 64

NC = 2
NS = 16
NW = NC * NS

NP = 10240
CH = 80
GP = 128

_MESH = dict(core_axis_name="c", subcore_axis_name="s")


def _wid():
    return lax.axis_index("s") * NC + lax.axis_index("c")


def _make_gather(B):
    assert B % (NW * CH) == 0
    chunks = B // (NW * CH)
    per_w = chunks * CH
    assert chunks % 2 == 0
    nh = chunks // 2

    @functools.partial(
        pl.kernel,
        out_type=jax.ShapeDtypeStruct((B, H), jnp.float32),
        mesh=plsc.VectorSubcoreMesh(**_MESH),
        scratch_types=[
            pltpu.VMEM((per_w,), jnp.int32),
            pltpu.VMEM((CH, H), jnp.float32),
            pltpu.VMEM((CH, H), jnp.float32),
            pltpu.SemaphoreType.DMA,
            pltpu.SemaphoreType.DMA,
        ],
    )
    def gather_k(table, idx, out, idx_v, rb0, rb1, s0, s1):
        base = _wid() * per_w
        pltpu.sync_copy(idx.at[pl.ds(base, per_w)], idx_v)

        def start(i, rb, s):
            pltpu.async_copy(table.at[idx_v.at[pl.ds(i * CH, CH)]], rb, s)

        def wait(rb, s):
            pltpu.make_async_copy(
                table.at[idx_v.at[pl.ds(0, CH)]], rb, s
            ).wait()

        start(0, rb0, s0)

        @pl.loop(0, nh)
        def _(j):
            i0 = 2 * j
            start(i0 + 1, rb1, s1)
            wait(rb0, s0)
            pltpu.sync_copy(rb0, out.at[pl.ds(base + i0 * CH, CH)])

            @pl.when(j < nh - 1)
            def _():
                start(i0 + 2, rb0, s0)

            wait(rb1, s1)
            pltpu.sync_copy(rb1, out.at[pl.ds(base + (i0 + 1) * CH, CH)])

    return gather_k


def _make_scatter(B, nseg):
    assert B % (NW * CH) == 0 and nseg % (NS * 8) == 0
    chunks = B // (NW * CH)
    per_w = chunks * CH
    nh = chunks // 2
    rem = chunks % 2
    rpt = nseg // NS

    @functools.partial(
        pl.kernel,
        out_type=jax.ShapeDtypeStruct((NC, nseg, H), jnp.float32),
        mesh=plsc.VectorSubcoreMesh(**_MESH),
        scratch_types=[
            pltpu.VMEM((CH,), jnp.int32),
            pltpu.VMEM((CH,), jnp.int32),
            pltpu.VMEM((CH, H), jnp.float32),
            pltpu.VMEM((CH, H), jnp.float32),
            pltpu.VMEM_SHARED((nseg, H), jnp.float32),
            pltpu.SemaphoreType.DMA,
            pltpu.SemaphoreType.DMA,
        ],
    )
    def scatter_k(vals, idx, zeros_c, out, ib0, ib1, vb0, vb1, acc, s0, s1):
        cid = lax.axis_index("c")
        sid = lax.axis_index("s")
        rbase = sid * rpt
        pltpu.sync_copy(zeros_c.at[pl.ds(0, CH)], vb0)
        nz = (rpt + CH - 1) // CH
        for j in range(nz):
            rows = min(CH, rpt - j * CH)
            pltpu.sync_copy(
                vb0.at[pl.ds(0, rows)], acc.at[pl.ds(rbase + j * CH, rows)]
            )
        plsc.subcore_barrier()

        base = _wid() * per_w

        def ld_idx(i, ib):
            pltpu.sync_copy(idx.at[pl.ds(base + i * CH, CH)], ib)

        def start(i, vb, s):
            pltpu.async_copy(vals.at[pl.ds(base + i * CH, CH)], vb, s)

        def wait(i, vb, s):
            pltpu.make_async_copy(vals.at[pl.ds(base, CH)], vb, s).wait()

        ld_idx(0, ib0)
        start(0, vb0, s0)

        @pl.loop(0, nh)
        def _(j):
            i0 = 2 * j
            ld_idx(i0 + 1, ib1)
            start(i0 + 1, vb1, s1)
            wait(i0, vb0, s0)
            pltpu.sync_copy(vb0, acc.at[ib0], add=True)

            @pl.when(j < nh - 1)
            def _():
                ld_idx(i0 + 2, ib0)
                start(i0 + 2, vb0, s0)

            wait(i0 + 1, vb1, s1)
            pltpu.sync_copy(vb1, acc.at[ib1], add=True)

        if rem:
            i_last = chunks - 1
            ld_idx(i_last, ib0)
            pltpu.sync_copy(vals.at[pl.ds(base + i_last * CH, CH)], vb0)
            pltpu.sync_copy(vb0, acc.at[ib0], add=True)

        plsc.subcore_barrier()
        pltpu.sync_copy(
            acc.at[pl.ds(rbase, rpt)], out.at[cid, pl.ds(rbase, rpt)]
        )

    return scatter_k


def _make_pool():
    chunks = NP // (NW * CH)
    per_w = chunks * CH
    rpt = GP // NS

    @functools.partial(
        pl.kernel,
        out_type=(
            jax.ShapeDtypeStruct((NC, GP, H), jnp.float32),
            jax.ShapeDtypeStruct((NC, GP, H), jnp.float32),
        ),
        mesh=plsc.VectorSubcoreMesh(**_MESH),
        scratch_types=[
            pltpu.VMEM((CH,), jnp.int32),
            pltpu.VMEM((CH, H), jnp.float32),
            pltpu.VMEM((CH, H), jnp.float32),
            pltpu.VMEM_SHARED((GP, H), jnp.float32),
            pltpu.VMEM_SHARED((GP, H), jnp.float32),
            pltpu.SemaphoreType.DMA,
        ],
    )
    def pool_k(vals, idx, zeros_c, ones_c, out_s, out_n, idx_v, vals_v,
               ones_v, acc_s, acc_n, sem):
        cid = lax.axis_index("c")
        sid = lax.axis_index("s")
        rbase = sid * rpt
        pltpu.sync_copy(zeros_c.at[pl.ds(0, CH)], vals_v)
        pltpu.sync_copy(ones_c.at[pl.ds(0, CH)], ones_v)
        pltpu.sync_copy(vals_v.at[pl.ds(0, rpt)], acc_s.at[pl.ds(rbase, rpt)])
        pltpu.sync_copy(vals_v.at[pl.ds(0, rpt)], acc_n.at[pl.ds(rbase, rpt)])
        plsc.subcore_barrier()

        base = _wid() * per_w

        @pl.loop(0, chunks)
        def _(i):
            off = base + i * CH
            pltpu.sync_copy(idx.at[pl.ds(off, CH)], idx_v)
            pltpu.sync_copy(vals.at[pl.ds(off, CH)], vals_v)
            pltpu.sync_copy(vals_v, acc_s.at[idx_v], add=True)
            pltpu.sync_copy(ones_v, acc_n.at[idx_v], add=True)

        plsc.subcore_barrier()
        pltpu.sync_copy(acc_s.at[pl.ds(rbase, rpt)], out_s.at[cid, pl.ds(rbase, rpt)])
        pltpu.sync_copy(acc_n.at[pl.ds(rbase, rpt)], out_n.at[cid, pl.ds(rbase, rpt)])

    return pool_k


CE = 640


def _edge_tc(hdhs, ea, Wc, bc):
    nb = E // CE

    def body(hd_ref, hs_ref, ea_ref, w_ref, b_ref, m_ref):
        w = w_ref[...]
        acc = jnp.dot(hd_ref[...], w[0:H], preferred_element_type=jnp.float32)
        acc += jnp.dot(hs_ref[...], w[H:2 * H], preferred_element_type=jnp.float32)
        acc += jnp.dot(ea_ref[...], w[2 * H:], preferred_element_type=jnp.float32)
        acc += b_ref[...]
        f = acc[:, :H]
        s = acc[:, H:]
        sig = 1.0 / (1.0 + jnp.exp(-f))
        sp = jnp.maximum(s, 0.0) + jnp.log1p(jnp.exp(-jnp.abs(s)))
        m_ref[...] = sig * sp

    return pl.pallas_call(
        body,
        grid=(nb,),
        in_specs=[
            pl.BlockSpec((CE, H), lambda i: (i, 0)),
            pl.BlockSpec((CE, H), lambda i: (i + nb, 0)),
            pl.BlockSpec((CE, R), lambda i: (i, 0)),
            pl.BlockSpec((2 * H + R, 2 * H), lambda i: (0, 0)),
            pl.BlockSpec((1, 2 * H), lambda i: (0, 0)),
        ],
        out_specs=pl.BlockSpec((CE, H), lambda i: (i, 0)),
        out_shape=jax.ShapeDtypeStruct((E, H), jnp.float32),
    )(hdhs, hdhs, ea, Wc, bc)


CB = 1024


def _combine_tc(h, parts, out_dtype):
    nb = NP // CB

    def body(h_ref, p0_ref, p1_ref, o_ref):
        hv = h_ref[...].astype(jnp.float32)
        o_ref[...] = jnp.maximum(hv + p0_ref[0] + p1_ref[0], 0.0).astype(out_dtype)

    return pl.pallas_call(
        body,
        grid=(nb,),
        in_specs=[
            pl.BlockSpec((CB, H), lambda i: (i, 0)),
            pl.BlockSpec((1, CB, H), lambda i: (0, i, 0)),
            pl.BlockSpec((1, CB, H), lambda i: (1, i, 0)),
        ],
        out_specs=pl.BlockSpec((CB, H), lambda i: (i, 0)),
        out_shape=jax.ShapeDtypeStruct((NP, H), out_dtype),
    )(h, parts, parts)


def _final_tc(sums, cnts, Wl, bl):
    def body(s_ref, c_ref, w_ref, b_ref, o_ref):
        ssum = s_ref[0] + s_ref[1]
        cnt = c_ref[0] + c_ref[1]
        pooled = ssum / jnp.maximum(cnt, 1.0)
        o_ref[...] = (
            jnp.dot(pooled, w_ref[...], preferred_element_type=jnp.float32)
            + b_ref[...]
        )

    return pl.pallas_call(
        body,
        in_specs=[
            pl.BlockSpec((NC, GP, H), lambda: (0, 0, 0)),
            pl.BlockSpec((NC, GP, H), lambda: (0, 0, 0)),
            pl.BlockSpec((H, H), lambda: (0, 0)),
            pl.BlockSpec((1, H), lambda: (0, 0)),
        ],
        out_specs=pl.BlockSpec((GP, H), lambda: (0, 0)),
        out_shape=jax.ShapeDtypeStruct((GP, H), jnp.float32),
    )(sums, cnts, Wl, bl.reshape(1, H))


_gather_emb = _make_gather(NP)
_gather_edges = _make_gather(2 * E)
_scatter_edges = _make_scatter(E, NP)
_pool = _make_pool()


def kernel(x, edge_index, edge_attr, batch, emb, Wf1, bf1, Ws1, bs1, Wf2, bf2,
           Ws2, bs2, Wf3, bf3, Ws3, bs3, Wl, bl):
    x = x.astype(jnp.int32)
    src = edge_index[0].astype(jnp.int32)
    dst = edge_index[1].astype(jnp.int32)
    batch = batch.astype(jnp.int32)

    xpad = jnp.pad(x, (0, NP - N))
    bpad = jnp.pad(batch, (0, NP - N), constant_values=G)
    eidx = jnp.concatenate([dst, src])
    zeros_c = jnp.zeros((CH, H), jnp.float32)
    ones_c = jnp.ones((CH, H), jnp.float32)
    h = _gather_emb(emb, xpad)

    layers = ((Wf1, bf1, Ws1, bs1), (Wf2, bf2, Ws2, bs2), (Wf3, bf3, Ws3, bs3))
    for Wf, bf, Ws, bs in layers:
        Wc = jnp.concatenate([Wf, Ws], axis=1)
        bc = jnp.concatenate([bf, bs]).reshape(1, 2 * H)
        hdhs = _gather_edges(h, eidx)
        m = _edge_tc(hdhs, edge_attr, Wc, bc)
        parts = _scatter_edges(m, dst, zeros_c)
        h = _combine_tc(h, parts, jnp.float32)

    sums, cnts = _pool(h, bpad, zeros_c, ones_c)
    out = _final_tc(sums, cnts, Wl, bl)
    return out[:G]
